# Initial kernel scaffold; baseline (speedup 1.0000x reference)
import jax
import jax.numpy as jnp
from jax.experimental import pallas as pl


def kernel(x, edge_index, hyperedge_index, gcn1_W, gcn1_b, gcn2_W, gcn2_b, hg1_W, hg1_b, hg2_W, hg2_b, num_iter):
    raise NotImplementedError("write your pallas kernel here")


# speedup vs baseline: 5.4486x; 5.4486x over previous
import functools

import jax
import jax.numpy as jnp
from jax import lax
from jax.experimental import pallas as pl
from jax.experimental.pallas import tpu as pltpu
from jax.experimental.pallas import tpu_sc as plsc

N = 10000
NHE = 10000
E = 320000
P = 320000
NFEAT = 128
NHID = 128
NOUT = 64
K = 10
TEMP = 30.0

NC = 2
NS = 16
NW = NC * NS

_MESH = dict(core_axis_name="c", subcore_axis_name="s", num_cores=NC,
             num_subcores=NS)


_NPAD = ((N + 8 * NS - 1) // (8 * NS)) * (8 * NS)


def _hist3(idx_a, idx_b, idx_c):
    CH = 80
    per_w = E // NW
    steps = per_w // CH
    rps = _NPAD // NS

    @functools.partial(
        pl.kernel,
        out_type=jax.ShapeDtypeStruct((NC, 3, _NPAD, 128), jnp.float32),
        mesh=plsc.VectorSubcoreMesh(**_MESH),
        scratch_types=[
            pltpu.VMEM((CH,), jnp.int32),
            pltpu.VMEM((CH, 128), jnp.float32),
            pltpu.VMEM_SHARED((_NPAD, 128), jnp.float32),
        ],
    )
    def k(a_hbm, b_hbm, c_hbm, zeros_hbm, ones_hbm, out_hbm, idx_v, ones_v,
          acc):
        c = lax.axis_index("c")
        s = lax.axis_index("s")
        w = s * NC + c
        base = w * per_w
        sl = pl.ds(s * rps, rps)
        pltpu.sync_copy(ones_hbm, ones_v)

        def do(j, src_hbm):
            pltpu.sync_copy(zeros_hbm.at[sl], acc.at[sl])
            plsc.subcore_barrier()

            def body(t, carry):
                pltpu.sync_copy(src_hbm.at[pl.ds(base + t * CH, CH)], idx_v)
                pltpu.sync_copy(ones_v, acc.at[idx_v], add=True)
                return carry

            lax.fori_loop(0, steps, body, 0)
            plsc.subcore_barrier()
            pltpu.sync_copy(acc.at[sl], out_hbm.at[c, j, sl])

        do(0, a_hbm)
        do(1, b_hbm)
        do(2, c_hbm)

    zeros = jnp.zeros((_NPAD, 128), jnp.float32)
    ones = jnp.ones((CH, 128), jnp.float32)
    return k(idx_a, idx_b, idx_c, zeros, ones)


def _make_segsum(F, n_edges, n_dst):
    CH = 80
    per_w = n_edges // NW
    steps = per_w // CH
    n_pad = ((n_dst + 8 * NS - 1) // (8 * NS)) * (8 * NS)
    rps = n_pad // NS

    @functools.partial(
        pl.kernel,
        out_type=jax.ShapeDtypeStruct((NC, n_pad, F), jnp.float32),
        mesh=plsc.VectorSubcoreMesh(**_MESH),
        scratch_types=[
            pltpu.VMEM((CH,), jnp.int32),
            pltpu.VMEM((CH,), jnp.int32),
            pltpu.VMEM((CH, F), jnp.float32),
            pltpu.VMEM_SHARED((n_pad, F), jnp.float32),
            pltpu.SemaphoreType.DMA,
        ],
    )
    def k(vals_hbm, src_hbm, dst_hbm, zeros_hbm, out_hbm, idx_s, idx_d, rows,
          acc, sem):
        c = lax.axis_index("c")
        s = lax.axis_index("s")
        w = s * NC + c
        pltpu.sync_copy(zeros_hbm.at[pl.ds(s * rps, rps)],
                        acc.at[pl.ds(s * rps, rps)])
        plsc.subcore_barrier()
        base = w * per_w

        def body(t, carry):
            off = base + t * CH
            pltpu.sync_copy(src_hbm.at[pl.ds(off, CH)], idx_s)
            pltpu.async_copy(vals_hbm.at[idx_s], rows, sem).wait()
            pltpu.sync_copy(dst_hbm.at[pl.ds(off, CH)], idx_d)
            pltpu.sync_copy(rows, acc.at[idx_d], add=True)
            return carry

        lax.fori_loop(0, steps, body, 0)
        plsc.subcore_barrier()
        pltpu.sync_copy(acc.at[pl.ds(s * rps, rps)],
                        out_hbm.at[c, pl.ds(s * rps, rps)])

    def run(vals, src_idx, dst_idx):
        zeros = jnp.zeros((n_pad, F), jnp.float32)
        return k(vals, src_idx, dst_idx, zeros)

    return run


BN = 2000
_G = N // BN


def _rows(block_shape):
    nlead = len(block_shape) - 2
    return pl.BlockSpec(block_shape,
                        lambda i, _n=nlead: (0,) * _n + (i, 0))


def _full(shape):
    return pl.BlockSpec(shape, lambda i: (0,) * len(shape))


def _kA(x_ref, w1_ref, whg_ref, bhg_ref, cnt_ref, h1_ref, hs1_ref, hh1_ref,
        dinv_ref, einv_ref, vinv_ref):
    x = x_ref[...]
    cnt = cnt_ref[...]
    deg = cnt[0, 0] + cnt[1, 0] + 1.0
    dinv = 1.0 / jnp.sqrt(deg)
    einv = 1.0 / jnp.maximum(cnt[0, 1] + cnt[1, 1], 1.0)
    vinv = 1.0 / jnp.maximum(cnt[0, 2] + cnt[1, 2], 1.0)
    h1 = jnp.dot(x, w1_ref[...], preferred_element_type=jnp.float32,
                 precision=lax.Precision.HIGHEST)
    hh1 = jnp.dot(x, whg_ref[...], preferred_element_type=jnp.float32,
                 precision=lax.Precision.HIGHEST)
    h1_ref[...] = h1
    hs1_ref[...] = dinv * h1
    hh1_ref[...] = hh1 + bhg_ref[...][None, :]
    dinv_ref[...] = dinv
    einv_ref[...] = einv
    vinv_ref[...] = vinv


def _kB(seg1_ref, h1_ref, dinv_ref, b1_ref, w2_ref, acce_ref, einv_ref,
        h2_ref, hs2_ref, efeat_ref):
    seg1 = seg1_ref[...]
    dinv = dinv_ref[...]
    h1 = h1_ref[...]
    x1 = dinv * (seg1[0] + seg1[1]) + dinv * dinv * h1 + b1_ref[...][None, :]
    x1 = jnp.maximum(x1, 0.0)
    h2 = jnp.dot(x1, w2_ref[...], preferred_element_type=jnp.float32,
                 precision=lax.Precision.HIGHEST)
    h2_ref[...] = h2
    hs2_ref[...] = jnp.concatenate([dinv * h2, jnp.zeros_like(h2)], axis=1)
    acce = acce_ref[...]
    efeat_ref[...] = (acce[0] + acce[1]) * einv_ref[...]


def _kC(seg2_ref, h2_ref, dinv_ref, b2_ref, accv_ref, vinv_ref, whg2_ref,
        bhg2_ref, x2_ref, hh2_ref):
    seg2 = seg2_ref[...][:, :, 0:NOUT]
    dinv = dinv_ref[...]
    h2 = h2_ref[...]
    x2 = dinv * (seg2[0] + seg2[1]) + dinv * dinv * h2 + b2_ref[...][None, :]
    accv = accv_ref[...]
    x3 = jnp.maximum((accv[0] + accv[1]) * vinv_ref[...], 0.0)
    hh2 = jnp.dot(x3, whg2_ref[...], preferred_element_type=jnp.float32,
                 precision=lax.Precision.HIGHEST)
    x2_ref[...] = x2
    hh2 = hh2 + bhg2_ref[...][None, :]
    hh2_ref[...] = jnp.concatenate([hh2, jnp.zeros_like(hh2)], axis=1)


def _kD(acce_ref, einv_ref, efeat_ref):
    acce = acce_ref[...]
    efeat_ref[...] = (acce[0] + acce[1]) * einv_ref[...]


def _kF(accv_ref, vinv_ref, x2_ref, emb_ref):
    accv = accv_ref[...][:, :, 0:NOUT]
    x4 = (accv[0] + accv[1]) * vinv_ref[...]
    emb_ref[...] = 0.5 * x2_ref[...] + 0.5 * x4


def _kE(niter_ref, emb_ref, mu_ref, r_ref, dist_ref):
    embeds = emb_ref[...]
    nrm = jnp.sqrt(jnp.sum(embeds * embeds, axis=1, keepdims=True))
    data = embeds / nrm
    hp = lax.Precision.HIGHEST

    d2 = jnp.sum((data - data[0:1, :]) ** 2, axis=1, keepdims=True)
    rows = lax.broadcasted_iota(jnp.int32, (N, 1), 0)
    centers = [data[0:1, :]]
    for _ in range(1, K):
        m = jnp.max(d2)
        cand = jnp.where(d2 == m, rows, N)
        imin = jnp.min(cand)
        onehot = (rows == imin).astype(jnp.float32)
        c = jnp.sum(data * onehot, axis=0, keepdims=True)
        centers.append(c)
        d2 = jnp.minimum(d2, jnp.sum((data - c) ** 2, axis=1, keepdims=True))
    mu0 = jnp.concatenate(centers, axis=0)

    onesN = jnp.ones((N, 1), jnp.float32)

    def soft_assign(mu):
        dist = lax.dot_general(data, mu, (((1,), (1,)), ((), ())),
                               precision=hp,
                               preferred_element_type=jnp.float32)
        z = TEMP * dist
        z = z - jnp.max(z, axis=1, keepdims=True)
        p = jnp.exp(z)
        r = p / jnp.sum(p, axis=1, keepdims=True)
        return dist, r

    def body(_, mu):
        _, r = soft_assign(mu)
        cm = lax.dot_general(r, data, (((0,), (0,)), ((), ())),
                             precision=hp,
                             preferred_element_type=jnp.float32)
        cr = lax.dot_general(r, onesN, (((0,), (0,)), ((), ())),
                             precision=hp,
                             preferred_element_type=jnp.float32)
        return cm / cr

    mu = lax.fori_loop(0, niter_ref[0], body, mu0)
    dist, r = soft_assign(mu)
    mu_ref[...] = mu
    r_ref[...] = r
    dist_ref[...] = dist


def kernel(x, edge_index, hyperedge_index, gcn1_W, gcn1_b, gcn2_W, gcn2_b,
           hg1_W, hg1_b, hg2_W, hg2_b, num_iter):
    src = edge_index[0]
    dst = edge_index[1]
    hv = hyperedge_index[0]
    he = hyperedge_index[1]

    counts = _hist3(dst, he, hv)[:, :, :, 0:1]

    f32 = jnp.float32
    h1, hs1, hh1, dinv, einv, vinv = pl.pallas_call(
        _kA,
        grid=(_G,),
        in_specs=[_rows((BN, NFEAT)), _full((NFEAT, NHID)),
                  _full((NFEAT, NHID)), _full((NHID,)),
                  _rows((NC, 3, BN, 1))],
        out_specs=[_rows((BN, NHID)), _rows((BN, NHID)), _rows((BN, NHID)),
                   _rows((BN, 1)), _rows((BN, 1)), _rows((BN, 1))],
        out_shape=[jax.ShapeDtypeStruct((N, NHID), f32),
                   jax.ShapeDtypeStruct((N, NHID), f32),
                   jax.ShapeDtypeStruct((N, NHID), f32),
                   jax.ShapeDtypeStruct((N, 1), f32),
                   jax.ShapeDtypeStruct((NHE, 1), f32),
                   jax.ShapeDtypeStruct((N, 1), f32)],
    )(x, gcn1_W, hg1_W, hg1_b, counts)

    segsum128 = _make_segsum(NHID, E, N)
    seg1 = segsum128(hs1, src, dst)
    acc_e1 = segsum128(hh1, hv, he)

    h2, hs2, efeat1 = pl.pallas_call(
        _kB,
        grid=(_G,),
        in_specs=[_rows((NC, BN, NHID)), _rows((BN, NHID)), _rows((BN, 1)),
                  _full((NHID,)), _full((NHID, NOUT)),
                  _rows((NC, BN, NHID)), _rows((BN, 1))],
        out_specs=[_rows((BN, NOUT)), _rows((BN, NHID)), _rows((BN, NHID))],
        out_shape=[jax.ShapeDtypeStruct((N, NOUT), f32),
                   jax.ShapeDtypeStruct((N, NHID), f32),
                   jax.ShapeDtypeStruct((NHE, NHID), f32)],
    )(seg1, h1, dinv, gcn1_b, gcn2_W, acc_e1, einv)

    seg2 = segsum128(hs2, src, dst)
    acc_v1 = segsum128(efeat1, he, hv)

    x2, hh2 = pl.pallas_call(
        _kC,
        grid=(_G,),
        in_specs=[_rows((NC, BN, NHID)), _rows((BN, NOUT)), _rows((BN, 1)),
                  _full((NOUT,)), _rows((NC, BN, NHID)), _rows((BN, 1)),
                  _full((NHID, NOUT)), _full((NOUT,))],
        out_specs=[_rows((BN, NOUT)), _rows((BN, NHID))],
        out_shape=[jax.ShapeDtypeStruct((N, NOUT), f32),
                   jax.ShapeDtypeStruct((N, NHID), f32)],
    )(seg2, h2, dinv, gcn2_b, acc_v1, vinv, hg2_W, hg2_b)

    acc_e2 = segsum128(hh2, hv, he)

    efeat2 = pl.pallas_call(
        _kD,
        grid=(_G,),
        in_specs=[_rows((NC, BN, NHID)), _rows((BN, 1))],
        out_specs=_rows((BN, NHID)),
        out_shape=jax.ShapeDtypeStruct((NHE, NHID), f32),
    )(acc_e2, einv)

    acc_v2 = segsum128(efeat2, he, hv)

    embeds = pl.pallas_call(
        _kF,
        grid=(_G,),
        in_specs=[_rows((NC, BN, NHID)), _rows((BN, 1)), _rows((BN, NOUT))],
        out_specs=_rows((BN, NOUT)),
        out_shape=jax.ShapeDtypeStruct((N, NOUT), f32),
    )(acc_v2, vinv, x2)

    niter = jnp.asarray(num_iter, jnp.int32).reshape((1,))
    mu, r, dist = pl.pallas_call(
        _kE,
        in_specs=[pl.BlockSpec(memory_space=pltpu.SMEM),
                  pl.BlockSpec((N, NOUT), lambda: (0, 0))],
        out_specs=[pl.BlockSpec((K, NOUT), lambda: (0, 0)),
                   pl.BlockSpec((N, K), lambda: (0, 0)),
                   pl.BlockSpec((N, K), lambda: (0, 0))],
        out_shape=[jax.ShapeDtypeStruct((K, NOUT), f32),
                   jax.ShapeDtypeStruct((N, K), f32),
                   jax.ShapeDtypeStruct((N, K), f32)],
    )(niter, embeds)

    return (mu, r, embeds, dist)


# speedup vs baseline: 8.7757x; 1.6106x over previous
import functools

import jax
import jax.numpy as jnp
from jax import lax
from jax.experimental import pallas as pl
from jax.experimental.pallas import tpu as pltpu
from jax.experimental.pallas import tpu_sc as plsc

N = 10000
NHE = 10000
E = 320000
P = 320000
NFEAT = 128
NHID = 128
NOUT = 64
K = 10
TEMP = 30.0

NC = 2
NS = 16
NW = NC * NS

_MESH = dict(core_axis_name="c", subcore_axis_name="s", num_cores=NC,
             num_subcores=NS)


_NPAD = ((N + 8 * NS - 1) // (8 * NS)) * (8 * NS)


def _hist3(idx_a, idx_b, idx_c):
    CH = 80
    per_w = E // NW
    steps = per_w // CH
    rps = _NPAD // NS

    @functools.partial(
        pl.kernel,
        out_type=jax.ShapeDtypeStruct((NC, 3, _NPAD, 128), jnp.float32),
        mesh=plsc.VectorSubcoreMesh(**_MESH),
        scratch_types=[
            pltpu.VMEM((CH,), jnp.int32),
            pltpu.VMEM((CH, 128), jnp.float32),
            pltpu.VMEM_SHARED((_NPAD, 128), jnp.float32),
        ],
    )
    def k(a_hbm, b_hbm, c_hbm, zeros_hbm, ones_hbm, out_hbm, idx_v, ones_v,
          acc):
        c = lax.axis_index("c")
        s = lax.axis_index("s")
        w = s * NC + c
        base = w * per_w
        sl = pl.ds(s * rps, rps)
        pltpu.sync_copy(ones_hbm, ones_v)

        def do(j, src_hbm):
            pltpu.sync_copy(zeros_hbm.at[sl], acc.at[sl])
            plsc.subcore_barrier()

            def body(t, carry):
                pltpu.sync_copy(src_hbm.at[pl.ds(base + t * CH, CH)], idx_v)
                pltpu.sync_copy(ones_v, acc.at[idx_v], add=True)
                return carry

            lax.fori_loop(0, steps, body, 0)
            plsc.subcore_barrier()
            pltpu.sync_copy(acc.at[sl], out_hbm.at[c, j, sl])

        do(0, a_hbm)
        do(1, b_hbm)
        do(2, c_hbm)

    zeros = jnp.zeros((_NPAD, 128), jnp.float32)
    ones = jnp.ones((CH, 128), jnp.float32)
    return k(idx_a, idx_b, idx_c, zeros, ones)


def _make_segsum(F, n_edges, n_dst):
    CH = 80
    per_w = n_edges // NW
    steps = per_w // CH
    n_pad = ((n_dst + 8 * NS - 1) // (8 * NS)) * (8 * NS)
    rps = n_pad // NS

    NB = 4
    assert (steps - 1) % NB == 0
    outer = (steps - 1) // NB

    @functools.partial(
        pl.kernel,
        out_type=jax.ShapeDtypeStruct((NC, n_pad, F), jnp.float32),
        mesh=plsc.VectorSubcoreMesh(**_MESH),
        scratch_types=[
            [pltpu.VMEM((CH,), jnp.int32)] * NB,
            [pltpu.VMEM((CH,), jnp.int32)] * NB,
            [pltpu.VMEM((CH, F), jnp.float32)] * NB,
            pltpu.VMEM_SHARED((n_pad, F), jnp.float32),
            pltpu.SemaphoreType.DMA,
            pltpu.SemaphoreType.DMA,
        ],
    )
    def k(vals_hbm, src_hbm, dst_hbm, zeros_hbm, out_hbm, idx_s, idx_d, rows,
          acc, gsem, ssem):
        c = lax.axis_index("c")
        s = lax.axis_index("s")
        w = s * NC + c
        pltpu.sync_copy(zeros_hbm.at[pl.ds(s * rps, rps)],
                        acc.at[pl.ds(s * rps, rps)])
        plsc.subcore_barrier()
        base = w * per_w

        def chunk_seq(off):
            pltpu.sync_copy(src_hbm.at[pl.ds(off, CH)], idx_s[0])
            pltpu.async_copy(vals_hbm.at[idx_s[0]], rows[0], gsem).wait()
            pltpu.sync_copy(dst_hbm.at[pl.ds(off, CH)], idx_d[0])
            pltpu.sync_copy(rows[0], acc.at[idx_d[0]], add=True)

        chunk_seq(base)

        def body(t, carry):
            off0 = base + (1 + t * NB) * CH
            gd = []
            for i in range(NB):
                off = off0 + i * CH
                pltpu.sync_copy(src_hbm.at[pl.ds(off, CH)], idx_s[i])
                gd.append(pltpu.async_copy(vals_hbm.at[idx_s[i]], rows[i],
                                           gsem))
            sd = []
            for i in range(NB):
                off = off0 + i * CH
                gd[i].wait()
                pltpu.sync_copy(dst_hbm.at[pl.ds(off, CH)], idx_d[i])
                sd.append(pltpu.async_copy(rows[i], acc.at[idx_d[i]], ssem,
                                           add=True))
            for i in range(NB):
                sd[i].wait()
            return carry

        lax.fori_loop(0, outer, body, 0)
        plsc.subcore_barrier()
        pltpu.sync_copy(acc.at[pl.ds(s * rps, rps)],
                        out_hbm.at[c, pl.ds(s * rps, rps)])

    def run(vals, src_idx, dst_idx):
        zeros = jnp.zeros((n_pad, F), jnp.float32)
        return k(vals, src_idx, dst_idx, zeros)

    return run


BN = 2000
_G = N // BN


def _rows(block_shape):
    nlead = len(block_shape) - 2
    return pl.BlockSpec(block_shape,
                        lambda i, _n=nlead: (0,) * _n + (i, 0))


def _full(shape):
    return pl.BlockSpec(shape, lambda i: (0,) * len(shape))


def _kA(x_ref, w1_ref, whg_ref, bhg_ref, cnt_ref, h1_ref, hs1_ref, hh1_ref,
        dinv_ref, einv_ref, vinv_ref):
    x = x_ref[...]
    cnt = cnt_ref[...]
    deg = cnt[0, 0] + cnt[1, 0] + 1.0
    dinv = 1.0 / jnp.sqrt(deg)
    einv = 1.0 / jnp.maximum(cnt[0, 1] + cnt[1, 1], 1.0)
    vinv = 1.0 / jnp.maximum(cnt[0, 2] + cnt[1, 2], 1.0)
    h1 = jnp.dot(x, w1_ref[...], preferred_element_type=jnp.float32,
                 precision=lax.Precision.HIGHEST)
    hh1 = jnp.dot(x, whg_ref[...], preferred_element_type=jnp.float32,
                 precision=lax.Precision.HIGHEST)
    h1_ref[...] = h1
    hs1_ref[...] = dinv * h1
    hh1_ref[...] = hh1 + bhg_ref[...][None, :]
    dinv_ref[...] = dinv
    einv_ref[...] = einv
    vinv_ref[...] = vinv


def _kB(seg1_ref, h1_ref, dinv_ref, b1_ref, w2_ref, acce_ref, einv_ref,
        h2_ref, hs2_ref, efeat_ref):
    seg1 = seg1_ref[...]
    dinv = dinv_ref[...]
    h1 = h1_ref[...]
    x1 = dinv * (seg1[0] + seg1[1]) + dinv * dinv * h1 + b1_ref[...][None, :]
    x1 = jnp.maximum(x1, 0.0)
    h2 = jnp.dot(x1, w2_ref[...], preferred_element_type=jnp.float32,
                 precision=lax.Precision.HIGHEST)
    h2_ref[...] = h2
    hs2_ref[...] = jnp.concatenate([dinv * h2, jnp.zeros_like(h2)], axis=1)
    acce = acce_ref[...]
    efeat_ref[...] = (acce[0] + acce[1]) * einv_ref[...]


def _kC(seg2_ref, h2_ref, dinv_ref, b2_ref, accv_ref, vinv_ref, whg2_ref,
        bhg2_ref, x2_ref, hh2_ref):
    seg2 = seg2_ref[...][:, :, 0:NOUT]
    dinv = dinv_ref[...]
    h2 = h2_ref[...]
    x2 = dinv * (seg2[0] + seg2[1]) + dinv * dinv * h2 + b2_ref[...][None, :]
    accv = accv_ref[...]
    x3 = jnp.maximum((accv[0] + accv[1]) * vinv_ref[...], 0.0)
    hh2 = jnp.dot(x3, whg2_ref[...], preferred_element_type=jnp.float32,
                 precision=lax.Precision.HIGHEST)
    x2_ref[...] = x2
    hh2 = hh2 + bhg2_ref[...][None, :]
    hh2_ref[...] = jnp.concatenate([hh2, jnp.zeros_like(hh2)], axis=1)


def _kD(acce_ref, einv_ref, efeat_ref):
    acce = acce_ref[...]
    efeat_ref[...] = (acce[0] + acce[1]) * einv_ref[...]


def _kF(accv_ref, vinv_ref, x2_ref, emb_ref):
    accv = accv_ref[...][:, :, 0:NOUT]
    x4 = (accv[0] + accv[1]) * vinv_ref[...]
    emb_ref[...] = 0.5 * x2_ref[...] + 0.5 * x4


def _kE(niter_ref, emb_ref, mu_ref, r_ref, dist_ref):
    embeds = emb_ref[...]
    nrm = jnp.sqrt(jnp.sum(embeds * embeds, axis=1, keepdims=True))
    data = embeds / nrm
    hp = lax.Precision.HIGHEST

    d2 = jnp.sum((data - data[0:1, :]) ** 2, axis=1, keepdims=True)
    rows = lax.broadcasted_iota(jnp.int32, (N, 1), 0)
    centers = [data[0:1, :]]
    for _ in range(1, K):
        m = jnp.max(d2)
        cand = jnp.where(d2 == m, rows, N)
        imin = jnp.min(cand)
        onehot = (rows == imin).astype(jnp.float32)
        c = jnp.sum(data * onehot, axis=0, keepdims=True)
        centers.append(c)
        d2 = jnp.minimum(d2, jnp.sum((data - c) ** 2, axis=1, keepdims=True))
    mu0 = jnp.concatenate(centers, axis=0)

    onesN = jnp.ones((N, 1), jnp.float32)

    def soft_assign(mu):
        dist = lax.dot_general(data, mu, (((1,), (1,)), ((), ())),
                               precision=hp,
                               preferred_element_type=jnp.float32)
        z = TEMP * dist
        z = z - jnp.max(z, axis=1, keepdims=True)
        p = jnp.exp(z)
        r = p / jnp.sum(p, axis=1, keepdims=True)
        return dist, r

    def body(_, mu):
        _, r = soft_assign(mu)
        cm = lax.dot_general(r, data, (((0,), (0,)), ((), ())),
                             precision=hp,
                             preferred_element_type=jnp.float32)
        cr = lax.dot_general(r, onesN, (((0,), (0,)), ((), ())),
                             precision=hp,
                             preferred_element_type=jnp.float32)
        return cm / cr

    mu = lax.fori_loop(0, niter_ref[0], body, mu0)
    dist, r = soft_assign(mu)
    mu_ref[...] = mu
    r_ref[...] = r
    dist_ref[...] = dist


def kernel(x, edge_index, hyperedge_index, gcn1_W, gcn1_b, gcn2_W, gcn2_b,
           hg1_W, hg1_b, hg2_W, hg2_b, num_iter):
    src = edge_index[0]
    dst = edge_index[1]
    hv = hyperedge_index[0]
    he = hyperedge_index[1]

    counts = _hist3(dst, he, hv)[:, :, :, 0:1]

    f32 = jnp.float32
    h1, hs1, hh1, dinv, einv, vinv = pl.pallas_call(
        _kA,
        grid=(_G,),
        in_specs=[_rows((BN, NFEAT)), _full((NFEAT, NHID)),
                  _full((NFEAT, NHID)), _full((NHID,)),
                  _rows((NC, 3, BN, 1))],
        out_specs=[_rows((BN, NHID)), _rows((BN, NHID)), _rows((BN, NHID)),
                   _rows((BN, 1)), _rows((BN, 1)), _rows((BN, 1))],
        out_shape=[jax.ShapeDtypeStruct((N, NHID), f32),
                   jax.ShapeDtypeStruct((N, NHID), f32),
                   jax.ShapeDtypeStruct((N, NHID), f32),
                   jax.ShapeDtypeStruct((N, 1), f32),
                   jax.ShapeDtypeStruct((NHE, 1), f32),
                   jax.ShapeDtypeStruct((N, 1), f32)],
    )(x, gcn1_W, hg1_W, hg1_b, counts)

    segsum128 = _make_segsum(NHID, E, N)
    seg1 = segsum128(hs1, src, dst)
    acc_e1 = segsum128(hh1, hv, he)

    h2, hs2, efeat1 = pl.pallas_call(
        _kB,
        grid=(_G,),
        in_specs=[_rows((NC, BN, NHID)), _rows((BN, NHID)), _rows((BN, 1)),
                  _full((NHID,)), _full((NHID, NOUT)),
                  _rows((NC, BN, NHID)), _rows((BN, 1))],
        out_specs=[_rows((BN, NOUT)), _rows((BN, NHID)), _rows((BN, NHID))],
        out_shape=[jax.ShapeDtypeStruct((N, NOUT), f32),
                   jax.ShapeDtypeStruct((N, NHID), f32),
                   jax.ShapeDtypeStruct((NHE, NHID), f32)],
    )(seg1, h1, dinv, gcn1_b, gcn2_W, acc_e1, einv)

    seg2 = segsum128(hs2, src, dst)
    acc_v1 = segsum128(efeat1, he, hv)

    x2, hh2 = pl.pallas_call(
        _kC,
        grid=(_G,),
        in_specs=[_rows((NC, BN, NHID)), _rows((BN, NOUT)), _rows((BN, 1)),
                  _full((NOUT,)), _rows((NC, BN, NHID)), _rows((BN, 1)),
                  _full((NHID, NOUT)), _full((NOUT,))],
        out_specs=[_rows((BN, NOUT)), _rows((BN, NHID))],
        out_shape=[jax.ShapeDtypeStruct((N, NOUT), f32),
                   jax.ShapeDtypeStruct((N, NHID), f32)],
    )(seg2, h2, dinv, gcn2_b, acc_v1, vinv, hg2_W, hg2_b)

    acc_e2 = segsum128(hh2, hv, he)

    efeat2 = pl.pallas_call(
        _kD,
        grid=(_G,),
        in_specs=[_rows((NC, BN, NHID)), _rows((BN, 1))],
        out_specs=_rows((BN, NHID)),
        out_shape=jax.ShapeDtypeStruct((NHE, NHID), f32),
    )(acc_e2, einv)

    acc_v2 = segsum128(efeat2, he, hv)

    embeds = pl.pallas_call(
        _kF,
        grid=(_G,),
        in_specs=[_rows((NC, BN, NHID)), _rows((BN, 1)), _rows((BN, NOUT))],
        out_specs=_rows((BN, NOUT)),
        out_shape=jax.ShapeDtypeStruct((N, NOUT), f32),
    )(acc_v2, vinv, x2)

    niter = jnp.asarray(num_iter, jnp.int32).reshape((1,))
    mu, r, dist = pl.pallas_call(
        _kE,
        in_specs=[pl.BlockSpec(memory_space=pltpu.SMEM),
                  pl.BlockSpec((N, NOUT), lambda: (0, 0))],
        out_specs=[pl.BlockSpec((K, NOUT), lambda: (0, 0)),
                   pl.BlockSpec((N, K), lambda: (0, 0)),
                   pl.BlockSpec((N, K), lambda: (0, 0))],
        out_shape=[jax.ShapeDtypeStruct((K, NOUT), f32),
                   jax.ShapeDtypeStruct((N, K), f32),
                   jax.ShapeDtypeStruct((N, K), f32)],
    )(niter, embeds)

    return (mu, r, embeds, dist)
